# baseline (speedup 1.0000x reference)
import jax
import jax.numpy as jnp
from jax import lax
from jax.experimental import pallas as pl
from jax.experimental.pallas import tpu as pltpu
from jax.experimental.pallas import tpu_sc as plsc

N = 320000
D = 128
H = 32
S = 4096

BLK = 12800
GRID_A = N // BLK

NC = 2
NS = 16
DC = D // NC
CPC = DC // 16
RPT = N // NS
RB = 400
NBLK = RPT // RB
G = RB // 16
SP = S + 16
SEG_PER_TILE = S // NS


def _scores_body(x_ref, w1_ref, b1_ref, w2_ref, s_ref, m_ref):
    i = pl.program_id(0)
    t = jnp.tanh(
        jnp.dot(x_ref[...], w1_ref[...], preferred_element_type=jnp.float32)
        + b1_ref[...]
    )
    s = lax.dot_general(w2_ref[...], t, (((1,), (1,)), ((), ())),
                        preferred_element_type=jnp.float32)
    s_ref[...] = s
    bm = jnp.max(s)

    @pl.when(i == 0)
    def _():
        m_ref[...] = jnp.full((1, 1), bm, jnp.float32)

    @pl.when(i > 0)
    def _():
        m_ref[...] = jnp.maximum(m_ref[...], bm)


_scores = pl.pallas_call(
    _scores_body,
    grid=(GRID_A,),
    in_specs=[
        pl.BlockSpec((BLK, D), lambda i: (i, 0)),
        pl.BlockSpec((D, H), lambda i: (0, 0)),
        pl.BlockSpec((1, H), lambda i: (0, 0)),
        pl.BlockSpec((1, H), lambda i: (0, 0)),
    ],
    out_specs=[
        pl.BlockSpec((1, BLK), lambda i: (0, i)),
        pl.BlockSpec((1, 1), lambda i: (0, 0)),
    ],
    out_shape=[
        jax.ShapeDtypeStruct((1, N), jnp.float32),
        jax.ShapeDtypeStruct((1, 1), jnp.float32),
    ],
)


def _sc_body(x_hbm, s_hbm, b_hbm, bf_hbm, m_hbm,
             out_hbm,
             x_v0, x_v1, s_v0, s_v1, i_v0, i_v1, f_v0, f_v1,
             accl, denl, segl, m_v, acc, den, sem_in, sem_sc):
    cid = lax.axis_index("c")
    sid = lax.axis_index("s")

    pltpu.sync_copy(m_hbm, m_v)

    def _zbody(r, _):
        for c in range(CPC):
            accl[r, pl.ds(c * 16, 16)] = jnp.zeros((16,), jnp.float32)
        denl[r, pl.ds(0, 16)] = jnp.zeros((16,), jnp.float32)
        return 0

    lax.fori_loop(0, SEG_PER_TILE, _zbody, 0)
    pltpu.sync_copy(accl.at[pl.ds(0, SEG_PER_TILE)],
                    acc.at[pl.ds(sid * SEG_PER_TILE, SEG_PER_TILE)])
    pltpu.sync_copy(denl.at[pl.ds(0, SEG_PER_TILE)],
                    den.at[pl.ds(sid * SEG_PER_TILE, SEG_PER_TILE)])
    plsc.subcore_barrier()

    mv = m_v[...]
    negf = jnp.full((16,), -1.0, jnp.float32)
    f_v0[pl.ds(0, 16)] = negf
    f_v1[pl.ds(0, 16)] = negf
    bufs = ((x_v0, s_v0, i_v0, f_v0), (x_v1, s_v1, i_v1, f_v1))

    def _in_start(j, b):
        base = sid * RPT + j * RB
        xb, sb, ib, fb = bufs[b]
        pltpu.async_copy(
            x_hbm.at[pl.ds(base, RB), pl.ds(cid * DC, DC)], xb, sem_in)
        pltpu.async_copy(s_hbm.at[0, pl.ds(base, RB)], sb, sem_in)
        pltpu.async_copy(b_hbm.at[pl.ds(base, RB)], ib, sem_in)
        pltpu.async_copy(bf_hbm.at[pl.ds(base, RB)], fb.at[pl.ds(8, RB)],
                         sem_in)

    def _in_wait(j, b):
        base = sid * RPT + j * RB
        xb, sb, ib, fb = bufs[b]
        pltpu.make_async_copy(
            x_hbm.at[pl.ds(base, RB), pl.ds(cid * DC, DC)], xb,
            sem_in).wait()
        pltpu.make_async_copy(s_hbm.at[0, pl.ds(base, RB)], sb,
                              sem_in).wait()
        pltpu.make_async_copy(b_hbm.at[pl.ds(base, RB)], ib, sem_in).wait()
        pltpu.make_async_copy(bf_hbm.at[pl.ds(base, RB)],
                              fb.at[pl.ds(8, RB)], sem_in).wait()

    def _process(j, b):
        xb, sb, ib, fb = bufs[b]

        dummy = jnp.full((16,), S, jnp.int32)

        def _slbody(t, _):
            segl[t, :] = dummy
            return 0

        lax.fori_loop(0, G, _slbody, 0)

        zero16 = jnp.zeros((16,), jnp.float32)
        one16 = jnp.full((16,), 1.0, jnp.float32)

        def _gbody(g, state):
            p = state[0]
            cd = state[1]
            carry = list(state[2:])
            ivf = fb[pl.ds(8 + g * 16, 16)]
            pvf = fb[pl.ds(7 + g * 16, 16)]
            dvf = ivf - pvf
            chf = jnp.minimum(dvf * dvf, one16)
            iv = ib[pl.ds(g * 16, 16)]
            pos_f = jnp.full((16,), p, jnp.float32) + plsc.cumsum(chf)
            pos = pos_f.astype(jnp.int32)
            plsc.store_scatter(segl, [pos >> 4, pos & 15], iv)
            ev = jnp.exp(sb[pl.ds(g * 16, 16)] - mv)
            for l in range(16):
                e = ev[l]
                cf = chf[l]
                r = g * 16 + l
                keep = 1.0 - cf
                for c in range(CPC):
                    tmp = xb[r, pl.ds(c * 16, 16)] * e
                    carry[c] = tmp + carry[c] * keep
                ebc = jnp.full((16,), e, jnp.float32)
                cd = ebc + cd * keep
                p = p + cf
                ip = p.astype(jnp.int32)
                for c in range(CPC):
                    accl[ip, pl.ds(c * 16, 16)] = carry[c]
                denl[ip, pl.ds(0, 16)] = cd
            return (p, cd, *carry)

        init = (jnp.float32(-1.0), zero16) + tuple(zero16 for _ in range(CPC))
        fin = lax.fori_loop(0, G, _gbody, init)
        n_used = fin[0].astype(jnp.int32) + 1

        for t in range(G):
            @pl.when(t * 16 < n_used)
            def _():
                pltpu.sync_copy(accl.at[pl.ds(t * 16, 16)],
                                acc.at[segl.at[t]], add=True)
                pltpu.sync_copy(denl.at[pl.ds(t * 16, 16)],
                                den.at[segl.at[t]], add=True)

    _in_start(0, 0)

    def _ibody(i, _):
        for b in range(2):
            j = 2 * i + b

            @pl.when(j + 1 < NBLK)
            def _():
                _in_start(j + 1, 1 - b)

            _in_wait(j, b)
            _process(j, b)
        return 0

    lax.fori_loop(0, NBLK // 2, _ibody, 0)
    plsc.subcore_barrier()

    pltpu.sync_copy(acc.at[pl.ds(sid * SEG_PER_TILE, SEG_PER_TILE)],
                    accl.at[pl.ds(0, SEG_PER_TILE)])
    pltpu.sync_copy(den.at[pl.ds(sid * SEG_PER_TILE, SEG_PER_TILE)],
                    denl.at[pl.ds(0, SEG_PER_TILE)])

    def _nbody(r, _):
        rec = 1.0 / (denl[r, pl.ds(0, 16)] + 1e-8)
        for c in range(CPC):
            accl[r, pl.ds(c * 16, 16)] = accl[r, pl.ds(c * 16, 16)] * rec
        return 0

    lax.fori_loop(0, SEG_PER_TILE, _nbody, 0)
    pltpu.sync_copy(accl.at[pl.ds(0, SEG_PER_TILE)],
                    out_hbm.at[pl.ds(sid * SEG_PER_TILE, SEG_PER_TILE),
                               pl.ds(cid * DC, DC)])


_sc_scatter = pl.kernel(
    _sc_body,
    out_type=jax.ShapeDtypeStruct((S, D), jnp.float32),
    mesh=plsc.VectorSubcoreMesh(core_axis_name="c", subcore_axis_name="s",
                                num_cores=NC, num_subcores=NS),
    scratch_types=[
        pltpu.VMEM((RB, DC), jnp.float32),
        pltpu.VMEM((RB, DC), jnp.float32),
        pltpu.VMEM((RB,), jnp.float32),
        pltpu.VMEM((RB,), jnp.float32),
        pltpu.VMEM((RB,), jnp.int32),
        pltpu.VMEM((RB,), jnp.int32),
        pltpu.VMEM((RB + 16,), jnp.float32),
        pltpu.VMEM((RB + 16,), jnp.float32),
        pltpu.VMEM((RB, DC), jnp.float32),
        pltpu.VMEM((RB, 16), jnp.float32),
        pltpu.VMEM((G, 16), jnp.int32),
        pltpu.VMEM((16,), jnp.float32),
        pltpu.VMEM_SHARED((SP, DC), jnp.float32),
        pltpu.VMEM_SHARED((SP, 16), jnp.float32),
        pltpu.SemaphoreType.DMA,
        pltpu.SemaphoreType.DMA,
    ],
    compiler_params=pltpu.CompilerParams(use_tc_tiling_on_sc=False,
                                         needs_layout_passes=False),
)


@jax.jit
def kernel(x, batch, W1, b1, W2, b2):
    del b2
    s, m = _scores(x, W1, b1.reshape(1, H), W2.reshape(1, H))
    m16 = jnp.broadcast_to(m.reshape(1), (16,))
    bi = batch.astype(jnp.int32)
    return _sc_scatter(x, s, bi, bi.astype(jnp.float32), m16)


# speedup vs baseline: 1.0078x; 1.0078x over previous
import jax
import jax.numpy as jnp
from jax import lax
from jax.experimental import pallas as pl
from jax.experimental.pallas import tpu as pltpu
from jax.experimental.pallas import tpu_sc as plsc

N = 320000
D = 128
H = 32
S = 4096

BLK = 12800
GRID_A = N // BLK

NC = 2
NS = 16
DC = D // NC
CPC = DC // 16
RPT = N // NS
RB = 400
NBLK = RPT // RB
G = RB // 16
SP = S + 16
SEG_PER_TILE = S // NS


def _scores_body(x_ref, w1_ref, b1_ref, w2_ref, s_ref, m_ref):
    i = pl.program_id(0)
    t = jnp.tanh(
        jnp.dot(x_ref[...], w1_ref[...], preferred_element_type=jnp.float32)
        + b1_ref[...]
    )
    s = lax.dot_general(w2_ref[...], t, (((1,), (1,)), ((), ())),
                        preferred_element_type=jnp.float32)
    s_ref[...] = s
    bm = jnp.max(s)

    @pl.when(i == 0)
    def _():
        m_ref[...] = jnp.full((1, 1), bm, jnp.float32)

    @pl.when(i > 0)
    def _():
        m_ref[...] = jnp.maximum(m_ref[...], bm)


_scores = pl.pallas_call(
    _scores_body,
    grid=(GRID_A,),
    in_specs=[
        pl.BlockSpec((BLK, D), lambda i: (i, 0)),
        pl.BlockSpec((D, H), lambda i: (0, 0)),
        pl.BlockSpec((1, H), lambda i: (0, 0)),
        pl.BlockSpec((1, H), lambda i: (0, 0)),
    ],
    out_specs=[
        pl.BlockSpec((1, BLK), lambda i: (0, i)),
        pl.BlockSpec((1, 1), lambda i: (0, 0)),
    ],
    out_shape=[
        jax.ShapeDtypeStruct((1, N), jnp.float32),
        jax.ShapeDtypeStruct((1, 1), jnp.float32),
    ],
)


def _sc_body(x_hbm, s_hbm, b_hbm, bf_hbm, m_hbm,
             out_hbm,
             x_v0, x_v1, s_v0, s_v1, i_v0, i_v1, f_v0, f_v1,
             accl, denl, segl, m_v, acc, den, sem_in, sem_sc):
    cid = lax.axis_index("c")
    sid = lax.axis_index("s")

    pltpu.sync_copy(m_hbm, m_v)

    def _zbody(r, _):
        for c in range(CPC):
            accl[r, pl.ds(c * 16, 16)] = jnp.zeros((16,), jnp.float32)
        denl[r, pl.ds(0, 16)] = jnp.zeros((16,), jnp.float32)
        return 0

    lax.fori_loop(0, SEG_PER_TILE, _zbody, 0)
    pltpu.sync_copy(accl.at[pl.ds(0, SEG_PER_TILE)],
                    acc.at[pl.ds(sid * SEG_PER_TILE, SEG_PER_TILE)])
    pltpu.sync_copy(denl.at[pl.ds(0, SEG_PER_TILE)],
                    den.at[pl.ds(sid * SEG_PER_TILE, SEG_PER_TILE)])
    plsc.subcore_barrier()

    mv = m_v[...]
    negf = jnp.full((16,), -1.0, jnp.float32)
    f_v0[pl.ds(0, 16)] = negf
    f_v1[pl.ds(0, 16)] = negf
    bufs = ((x_v0, s_v0, i_v0, f_v0), (x_v1, s_v1, i_v1, f_v1))

    def _in_start(j, b):
        base = sid * RPT + j * RB
        xb, sb, ib, fb = bufs[b]
        pltpu.async_copy(
            x_hbm.at[pl.ds(base, RB), pl.ds(cid * DC, DC)], xb, sem_in)
        pltpu.async_copy(s_hbm.at[0, pl.ds(base, RB)], sb, sem_in)
        pltpu.async_copy(b_hbm.at[pl.ds(base, RB)], ib, sem_in)
        pltpu.async_copy(bf_hbm.at[pl.ds(base, RB)], fb.at[pl.ds(8, RB)],
                         sem_in)

    def _in_wait(j, b):
        base = sid * RPT + j * RB
        xb, sb, ib, fb = bufs[b]
        pltpu.make_async_copy(
            x_hbm.at[pl.ds(base, RB), pl.ds(cid * DC, DC)], xb,
            sem_in).wait()
        pltpu.make_async_copy(s_hbm.at[0, pl.ds(base, RB)], sb,
                              sem_in).wait()
        pltpu.make_async_copy(b_hbm.at[pl.ds(base, RB)], ib, sem_in).wait()
        pltpu.make_async_copy(bf_hbm.at[pl.ds(base, RB)],
                              fb.at[pl.ds(8, RB)], sem_in).wait()

    def _process(j, b):
        xb, sb, ib, fb = bufs[b]

        dummy = jnp.full((16,), S, jnp.int32)

        def _slbody(t, _):
            segl[t, :] = dummy
            return 0

        lax.fori_loop(0, G, _slbody, 0)

        zero16 = jnp.zeros((16,), jnp.float32)
        one16 = jnp.full((16,), 1.0, jnp.float32)

        def _gbody(g, state):
            p = state[0]
            cd = state[1]
            carry = list(state[2:])
            ivf = fb[pl.ds(8 + g * 16, 16)]
            pvf = fb[pl.ds(7 + g * 16, 16)]
            dvf = ivf - pvf
            chf = jnp.minimum(dvf * dvf, one16)
            iv = ib[pl.ds(g * 16, 16)]
            pos_f = jnp.full((16,), p, jnp.float32) + plsc.cumsum(chf)
            pos = pos_f.astype(jnp.int32)
            plsc.store_scatter(segl, [pos >> 4, pos & 15], iv)
            ev = jnp.exp(sb[pl.ds(g * 16, 16)] - mv)
            for l in range(16):
                e = ev[l]
                cf = chf[l]
                r = g * 16 + l
                old_carry = list(carry)
                old_cd = cd
                old_ip = p.astype(jnp.int32)

                @pl.when((cf > 0.5) & (p >= 0.0))
                def _():
                    for c in range(CPC):
                        accl[old_ip, pl.ds(c * 16, 16)] = old_carry[c]
                    denl[old_ip, pl.ds(0, 16)] = old_cd

                keep = 1.0 - cf
                for c in range(CPC):
                    tmp = xb[r, pl.ds(c * 16, 16)] * e
                    carry[c] = tmp + carry[c] * keep
                ebc = jnp.full((16,), e, jnp.float32)
                cd = ebc + cd * keep
                p = p + cf
            return (p, cd, *carry)

        init = (jnp.float32(-1.0), zero16) + tuple(zero16 for _ in range(CPC))
        fin = lax.fori_loop(0, G, _gbody, init)
        pf = fin[0].astype(jnp.int32)
        for c in range(CPC):
            accl[pf, pl.ds(c * 16, 16)] = fin[2 + c]
        denl[pf, pl.ds(0, 16)] = fin[1]
        n_used = pf + 1

        for t in range(G):
            @pl.when(t * 16 < n_used)
            def _():
                pltpu.sync_copy(accl.at[pl.ds(t * 16, 16)],
                                acc.at[segl.at[t]], add=True)
                pltpu.sync_copy(denl.at[pl.ds(t * 16, 16)],
                                den.at[segl.at[t]], add=True)

    _in_start(0, 0)

    def _ibody(i, _):
        for b in range(2):
            j = 2 * i + b
            _in_wait(j, b)

            @pl.when(j + 1 < NBLK)
            def _():
                _in_start(j + 1, 1 - b)

            _process(j, b)
        return 0

    lax.fori_loop(0, NBLK // 2, _ibody, 0)
    plsc.subcore_barrier()

    pltpu.sync_copy(acc.at[pl.ds(sid * SEG_PER_TILE, SEG_PER_TILE)],
                    accl.at[pl.ds(0, SEG_PER_TILE)])
    pltpu.sync_copy(den.at[pl.ds(sid * SEG_PER_TILE, SEG_PER_TILE)],
                    denl.at[pl.ds(0, SEG_PER_TILE)])

    def _nbody(r, _):
        rec = 1.0 / (denl[r, pl.ds(0, 16)] + 1e-8)
        for c in range(CPC):
            accl[r, pl.ds(c * 16, 16)] = accl[r, pl.ds(c * 16, 16)] * rec
        return 0

    lax.fori_loop(0, SEG_PER_TILE, _nbody, 0)
    pltpu.sync_copy(accl.at[pl.ds(0, SEG_PER_TILE)],
                    out_hbm.at[pl.ds(sid * SEG_PER_TILE, SEG_PER_TILE),
                               pl.ds(cid * DC, DC)])


_sc_scatter = pl.kernel(
    _sc_body,
    out_type=jax.ShapeDtypeStruct((S, D), jnp.float32),
    mesh=plsc.VectorSubcoreMesh(core_axis_name="c", subcore_axis_name="s",
                                num_cores=NC, num_subcores=NS),
    scratch_types=[
        pltpu.VMEM((RB, DC), jnp.float32),
        pltpu.VMEM((RB, DC), jnp.float32),
        pltpu.VMEM((RB,), jnp.float32),
        pltpu.VMEM((RB,), jnp.float32),
        pltpu.VMEM((RB,), jnp.int32),
        pltpu.VMEM((RB,), jnp.int32),
        pltpu.VMEM((RB + 16,), jnp.float32),
        pltpu.VMEM((RB + 16,), jnp.float32),
        pltpu.VMEM((RB, DC), jnp.float32),
        pltpu.VMEM((RB, 16), jnp.float32),
        pltpu.VMEM((G, 16), jnp.int32),
        pltpu.VMEM((16,), jnp.float32),
        pltpu.VMEM_SHARED((SP, DC), jnp.float32),
        pltpu.VMEM_SHARED((SP, 16), jnp.float32),
        pltpu.SemaphoreType.DMA,
        pltpu.SemaphoreType.DMA,
    ],
    compiler_params=pltpu.CompilerParams(use_tc_tiling_on_sc=False,
                                         needs_layout_passes=False),
)


@jax.jit
def kernel(x, batch, W1, b1, W2, b2):
    del b2
    s, m = _scores(x, W1, b1.reshape(1, H), W2.reshape(1, H))
    m16 = jnp.broadcast_to(m.reshape(1), (16,))
    bi = batch.astype(jnp.int32)
    return _sc_scatter(x, s, bi, bi.astype(jnp.float32), m16)
